# baseline (speedup 1.0000x reference)
import functools

import jax
import jax.numpy as jnp
from jax import lax
from jax.experimental import pallas as pl
from jax.experimental.pallas import tpu as pltpu
from jax.experimental.pallas import tpu_sc as plsc

VOCAB = 100000
D = 128
B = 16384
NC, NS, L = 2, 16, 16
NW = NC * NS
BPW = B // NW
CHUNK = 64
NCHUNK = BPW // CHUNK

_mesh = plsc.VectorSubcoreMesh(
    core_axis_name="c", subcore_axis_name="s", num_cores=NC, num_subcores=NS
)


@functools.partial(
    pl.kernel,
    out_type=jax.ShapeDtypeStruct((NW, NCHUNK, CHUNK, D), jnp.float32),
    mesh=_mesh,
    scratch_types=[
        pltpu.VMEM((BPW,), jnp.int32),
        pltpu.VMEM((NCHUNK, CHUNK), jnp.int32),
        pltpu.VMEM((NCHUNK, CHUNK, D), jnp.float32),
        pltpu.SemaphoreType.DMA((NCHUNK,)),
        pltpu.SemaphoreType.DMA,
    ],
)
def _gather_kernel(label_hbm, table_hbm, out_hbm, lbl_v, idx_v, rows_v, gsem, osem):
    wid = lax.axis_index("s") * NC + lax.axis_index("c")
    pltpu.sync_copy(label_hbm.at[pl.ds(wid * BPW, BPW)], lbl_v)

    def compute_idx(j):
        for i in range(CHUNK // L):
            v = lbl_v[pl.ds(j * CHUNK + i * L, L)]
            idx_v[j, pl.ds(i * L, L)] = v * 2 + 1

    def fire_gather(j):
        return pltpu.async_copy(table_hbm.at[idx_v.at[j]], rows_v.at[j], gsem.at[j])

    compute_idx(0)
    gathers = [fire_gather(0)]
    outs = []
    for j in range(1, NCHUNK):
        compute_idx(j)
        gathers.append(fire_gather(j))
        gathers[j - 1].wait()
        outs.append(pltpu.async_copy(rows_v.at[j - 1], out_hbm.at[wid, j - 1], osem))
    gathers[NCHUNK - 1].wait()
    outs.append(
        pltpu.async_copy(rows_v.at[NCHUNK - 1], out_hbm.at[wid, NCHUNK - 1], osem)
    )
    for c in outs:
        c.wait()


def kernel(label, text_vectors):
    table = text_vectors.reshape(2 * VOCAB, D)
    out = _gather_kernel(label.astype(jnp.int32), table)
    return out.reshape(B, 1, D)


# speedup vs baseline: 1.0577x; 1.0577x over previous
import functools

import jax
import jax.numpy as jnp
from jax import lax
from jax.experimental import pallas as pl
from jax.experimental.pallas import tpu as pltpu
from jax.experimental.pallas import tpu_sc as plsc

VOCAB = 100000
D = 128
B = 16384
NC, NS, L = 2, 16, 16
NW = NC * NS
BPW = B // NW
CHUNK = 128
NCHUNK = BPW // CHUNK

_mesh = plsc.VectorSubcoreMesh(
    core_axis_name="c", subcore_axis_name="s", num_cores=NC, num_subcores=NS
)


@functools.partial(
    pl.kernel,
    out_type=jax.ShapeDtypeStruct((NW, NCHUNK, CHUNK, D), jnp.float32),
    mesh=_mesh,
    scratch_types=[
        pltpu.VMEM((BPW,), jnp.int32),
        pltpu.VMEM((NCHUNK, CHUNK), jnp.int32),
        pltpu.VMEM((NCHUNK, CHUNK, D), jnp.float32),
        pltpu.SemaphoreType.DMA((NCHUNK,)),
        pltpu.SemaphoreType.DMA,
    ],
)
def _gather_kernel(label_hbm, table_hbm, out_hbm, lbl_v, idx_v, rows_v, gsem, osem):
    wid = lax.axis_index("s") * NC + lax.axis_index("c")
    pltpu.sync_copy(label_hbm.at[pl.ds(wid * BPW, BPW)], lbl_v)
    for j in range(NCHUNK):
        for i in range(CHUNK // L):
            v = lbl_v[pl.ds(j * CHUNK + i * L, L)]
            idx_v[j, pl.ds(i * L, L)] = v * 2 + 1
    gathers = [
        pltpu.async_copy(table_hbm.at[idx_v.at[j]], rows_v.at[j], gsem.at[j])
        for j in range(NCHUNK)
    ]
    outs = []
    for j in range(NCHUNK):
        gathers[j].wait()
        outs.append(pltpu.async_copy(rows_v.at[j], out_hbm.at[wid, j], osem))
    for c in outs:
        c.wait()


def kernel(label, text_vectors):
    table = text_vectors.reshape(2 * VOCAB, D)
    out = _gather_kernel(label.astype(jnp.int32), table)
    return out.reshape(B, 1, D)
